# Initial kernel scaffold; baseline (speedup 1.0000x reference)
import jax
import jax.numpy as jnp
from jax.experimental import pallas as pl


def kernel(feat, graph_edge_index, edge_index, W1, b1, W2, b2, Wc, bc):
    raise NotImplementedError("write your pallas kernel here")


# speedup vs baseline: 4.6849x; 4.6849x over previous
import functools

import jax
import jax.numpy as jnp
from jax import lax
from jax.experimental import pallas as pl
from jax.experimental.pallas import tpu as pltpu
from jax.experimental.pallas import tpu_sc as plsc

N_NODES = 10000
N_EDGES = 320000
N_CLS = 100000
D = 128
OUT_DIM = 2

N_PAD = 10240
K = 128
NC, NS = 2, 16
NW = NC * NS

E_CHUNKS = 79
E_PAD = NW * E_CHUNKS * K
E_PER_CORE = E_PAD // NC
E_PER_TILE = E_PAD // NW

C_CHUNKS = 25
EC_PAD = NW * C_CHUNKS * K
EC_PER_TILE = EC_PAD // NW

ROWS_PER_TILE = N_PAD // NS
HW = 16

_mesh = plsc.VectorSubcoreMesh(core_axis_name="c", subcore_axis_name="s")
_sc_params = pltpu.CompilerParams(needs_layout_passes=False)
_sc_params_flat = pltpu.CompilerParams(
    needs_layout_passes=False, use_tc_tiling_on_sc=False)


@functools.partial(
    pl.kernel,
    out_type=jax.ShapeDtypeStruct((NC, 2 * N_PAD, HW), jnp.float32),
    mesh=_mesh,
    compiler_params=_sc_params_flat,
    scratch_types=[
        pltpu.VMEM((K,), jnp.int32),
        pltpu.VMEM((K, HW), jnp.float32),
        pltpu.VMEM_SHARED((2 * N_PAD, HW), jnp.float32),
    ],
)
def _degrees(src_hbm, dst_hbm, out_hbm, idx_v, ones_v, hist_sh):
    c = lax.axis_index("c")
    s = lax.axis_index("s")
    zero16 = jnp.zeros((HW,), jnp.float32)
    e0 = jnp.where(lax.iota(jnp.int32, HW) == 0, 1.0, 0.0).astype(jnp.float32)

    def fill(val):
        def body(i, _):
            ones_v[i, :] = val
            return 0
        lax.fori_loop(0, K, body, 0)

    fill(zero16)
    hrows = (2 * N_PAD) // NS
    for t in range(hrows // K):
        pltpu.sync_copy(ones_v, hist_sh.at[pl.ds(s * hrows + t * K, K)])
    fill(e0)
    plsc.subcore_barrier()

    def step(k, _):
        base = c * E_PER_CORE + s * E_PER_TILE + k * K
        pltpu.sync_copy(src_hbm.at[pl.ds(base, K)], idx_v)
        pltpu.sync_copy(ones_v, hist_sh.at[idx_v], add=True)
        pltpu.sync_copy(dst_hbm.at[pl.ds(base, K)], idx_v)
        for j in range(K // 16):
            idx_v[pl.ds(j * 16, 16)] = idx_v[pl.ds(j * 16, 16)] + N_PAD
        pltpu.sync_copy(ones_v, hist_sh.at[idx_v], add=True)
        return 0

    lax.fori_loop(0, E_CHUNKS, step, 0)
    plsc.subcore_barrier()
    pltpu.sync_copy(hist_sh.at[pl.ds(s * hrows, hrows)],
                    out_hbm.at[c, pl.ds(s * hrows, hrows)])


@functools.partial(
    pl.kernel,
    out_type=jax.ShapeDtypeStruct((NC, N_PAD, D), jnp.float32),
    mesh=_mesh,
    compiler_params=_sc_params,
    scratch_types=[
        pltpu.VMEM((K,), jnp.int32),
        pltpu.VMEM((K,), jnp.int32),
        pltpu.VMEM((K, D), jnp.float32),
        pltpu.VMEM_SHARED((N_PAD, D), jnp.float32),
        pltpu.SemaphoreType.DMA,
    ],
)
def _aggregate(x_hbm, src_hbm, dst_hbm, out_hbm, sidx, didx, rows, acc_sh, sem):
    c = lax.axis_index("c")
    s = lax.axis_index("s")
    zerov = jnp.zeros((16,), jnp.float32)

    def zrow(i, _):
        for j in range(D // 16):
            rows[i, pl.ds(j * 16, 16)] = zerov
        return 0

    lax.fori_loop(0, K, zrow, 0)
    base_r = s * ROWS_PER_TILE
    for t in range(ROWS_PER_TILE // K):
        pltpu.sync_copy(rows, acc_sh.at[pl.ds(base_r + t * K, K)])
    plsc.subcore_barrier()

    def step(k, _):
        base = c * E_PER_CORE + s * E_PER_TILE + k * K
        pltpu.sync_copy(src_hbm.at[pl.ds(base, K)], sidx)
        pltpu.async_copy(x_hbm.at[sidx], rows, sem).wait()
        pltpu.sync_copy(dst_hbm.at[pl.ds(base, K)], didx)
        pltpu.sync_copy(rows, acc_sh.at[didx], add=True)
        return 0

    lax.fori_loop(0, E_CHUNKS, step, 0)
    plsc.subcore_barrier()
    pltpu.sync_copy(acc_sh.at[pl.ds(base_r, ROWS_PER_TILE)],
                    out_hbm.at[c, pl.ds(base_r, ROWS_PER_TILE)])


@functools.partial(
    pl.kernel,
    out_type=jax.ShapeDtypeStruct((EC_PAD * OUT_DIM,), jnp.float32),
    mesh=_mesh,
    compiler_params=_sc_params,
    scratch_types=[
        pltpu.VMEM((N_PAD * 8,), jnp.float32),
        pltpu.VMEM((K,), jnp.int32),
        pltpu.VMEM((K,), jnp.int32),
        pltpu.VMEM((K * OUT_DIM,), jnp.float32),
    ],
)
def _classify(pq_hbm, s_hbm, d_hbm, out_hbm, pq_v, sidx, didx, stage):
    c = lax.axis_index("c")
    s = lax.axis_index("s")
    pltpu.sync_copy(pq_hbm, pq_v)
    lane = lax.iota(jnp.int32, 16)

    def step(k, _):
        base = (s * NC + c) * EC_PER_TILE + k * K
        pltpu.sync_copy(s_hbm.at[pl.ds(base, K)], sidx)
        pltpu.sync_copy(d_hbm.at[pl.ds(base, K)], didx)
        for j in range(K // 16):
            sv = sidx[pl.ds(j * 16, 16)] * 8
            dv = didx[pl.ds(j * 16, 16)] * 8
            p0 = plsc.load_gather(pq_v, [sv])
            p1 = plsc.load_gather(pq_v, [sv + 1])
            q0 = plsc.load_gather(pq_v, [dv + 2])
            q1 = plsc.load_gather(pq_v, [dv + 3])
            l0 = p0 + q0
            l1 = p1 + q1
            s0 = 1.0 / (1.0 + jnp.exp(-l0))
            s1 = 1.0 / (1.0 + jnp.exp(-l1))
            pos = j * 32 + lane * 2
            plsc.store_scatter(stage, [pos], s0)
            plsc.store_scatter(stage, [pos + 1], s1)
        pltpu.sync_copy(stage, out_hbm.at[pl.ds(base * OUT_DIM, K * OUT_DIM)])
        return 0

    lax.fori_loop(0, C_CHUNKS, step, 0)


BLK = 256
NB = N_PAD // BLK


def _prep_body(hist_ref, feat_ref, x1_ref):
    blk = hist_ref[...]
    deg = blk[0] + blk[1]
    ns = lax.rsqrt(jnp.maximum(deg[:, 0:1], 1.0))
    x1_ref[...] = feat_ref[...] * ns


_prep = pl.pallas_call(
    _prep_body,
    grid=(NB,),
    in_specs=[
        pl.BlockSpec((NC, BLK, HW), lambda i: (0, i, 0)),
        pl.BlockSpec((BLK, D), lambda i: (i, 0)),
    ],
    out_specs=pl.BlockSpec((BLK, D), lambda i: (i, 0)),
    out_shape=jax.ShapeDtypeStruct((N_PAD, D), jnp.float32),
)


def _layer1_body(agg_ref, hd_ref, hs_ref, w_ref, b_ref, out_ref):
    a = agg_ref[0] + agg_ref[1]
    dd = hd_ref[0] + hd_ref[1]
    nd = lax.rsqrt(jnp.maximum(dd[:, 0:1], 1.0))
    y = jnp.dot(a * nd, w_ref[...], preferred_element_type=jnp.float32)
    y = jnp.maximum(y + b_ref[...], 0.0)
    sd = hs_ref[0] + hs_ref[1]
    ns = lax.rsqrt(jnp.maximum(sd[:, 0:1], 1.0))
    out_ref[...] = y * ns


_layer1 = pl.pallas_call(
    _layer1_body,
    grid=(NB,),
    in_specs=[
        pl.BlockSpec((NC, BLK, D), lambda i: (0, i, 0)),
        pl.BlockSpec((NC, BLK, HW), lambda i: (0, NB + i, 0)),
        pl.BlockSpec((NC, BLK, HW), lambda i: (0, i, 0)),
        pl.BlockSpec((D, D), lambda i: (0, 0)),
        pl.BlockSpec((1, D), lambda i: (0, 0)),
    ],
    out_specs=pl.BlockSpec((BLK, D), lambda i: (i, 0)),
    out_shape=jax.ShapeDtypeStruct((N_PAD, D), jnp.float32),
)


def _layer2_body(agg_ref, hd_ref, w_ref, b_ref, wc_ref, bc_ref, h_ref, pq_ref):
    a = agg_ref[0] + agg_ref[1]
    dd = hd_ref[0] + hd_ref[1]
    nd = lax.rsqrt(jnp.maximum(dd[:, 0:1], 1.0))
    y = jnp.dot(a * nd, w_ref[...], preferred_element_type=jnp.float32)
    y = jnp.maximum(y + b_ref[...], 0.0)
    h_ref[...] = y
    pq_ref[...] = (
        jnp.dot(y, wc_ref[...], preferred_element_type=jnp.float32) + bc_ref[...]
    )


_layer2 = pl.pallas_call(
    _layer2_body,
    grid=(NB,),
    in_specs=[
        pl.BlockSpec((NC, BLK, D), lambda i: (0, i, 0)),
        pl.BlockSpec((NC, BLK, HW), lambda i: (0, NB + i, 0)),
        pl.BlockSpec((D, D), lambda i: (0, 0)),
        pl.BlockSpec((1, D), lambda i: (0, 0)),
        pl.BlockSpec((D, 8), lambda i: (0, 0)),
        pl.BlockSpec((1, 8), lambda i: (0, 0)),
    ],
    out_specs=[
        pl.BlockSpec((BLK, D), lambda i: (i, 0)),
        pl.BlockSpec((BLK, 8), lambda i: (i, 0)),
    ],
    out_shape=[
        jax.ShapeDtypeStruct((N_PAD, D), jnp.float32),
        jax.ShapeDtypeStruct((N_PAD, 8), jnp.float32),
    ],
)


def kernel(feat, graph_edge_index, edge_index, W1, b1, W2, b2, Wc, bc):
    f32 = jnp.float32
    feat_pad = jnp.pad(feat.astype(f32), ((0, N_PAD - N_NODES), (0, 0)))

    n_pad_e = E_PAD - N_EDGES
    pad_idx = (jnp.arange(n_pad_e, dtype=jnp.int32) % (N_PAD - N_NODES)) + N_NODES
    src_p = jnp.concatenate([graph_edge_index[0].astype(jnp.int32), pad_idx])
    dst_p = jnp.concatenate([graph_edge_index[1].astype(jnp.int32), pad_idx])

    n_pad_c = EC_PAD - N_CLS
    zpad = jnp.zeros((n_pad_c,), jnp.int32)
    cs_p = jnp.concatenate([edge_index[0].astype(jnp.int32), zpad])
    cd_p = jnp.concatenate([edge_index[1].astype(jnp.int32), zpad])

    b1r = b1.astype(f32).reshape(1, D)
    b2r = b2.astype(f32).reshape(1, D)
    wc8 = jnp.zeros((D, 8), f32)
    wc8 = wc8.at[:, 0:2].set(Wc[:D].astype(f32))
    wc8 = wc8.at[:, 2:4].set(Wc[D:].astype(f32))
    bc8 = jnp.zeros((1, 8), f32).at[0, 0:2].set(bc.astype(f32))

    hist = _degrees(src_p, dst_p)
    x1 = _prep(hist, feat_pad)
    agg1 = _aggregate(x1, src_p, dst_p)
    x2 = _layer1(agg1, hist, hist, W1.astype(f32), b1r)
    agg2 = _aggregate(x2, src_p, dst_p)
    h, pq = _layer2(agg2, hist, W2.astype(f32), b2r, wc8, bc8)
    probs = _classify(pq.reshape(N_PAD * 8), cs_p, cd_p)

    return h[:N_NODES], probs.reshape(EC_PAD, OUT_DIM)[:N_CLS]


# speedup vs baseline: 7.5845x; 1.6189x over previous
import functools

import jax
import jax.numpy as jnp
from jax import lax
from jax.experimental import pallas as pl
from jax.experimental.pallas import tpu as pltpu
from jax.experimental.pallas import tpu_sc as plsc

N_NODES = 10000
N_EDGES = 320000
N_CLS = 100000
D = 128
OUT_DIM = 2

N_PAD = 10240
K = 128
NC, NS = 2, 16
NW = NC * NS

E_CHUNKS = 79
E_PAD = NW * E_CHUNKS * K
E_PER_CORE = E_PAD // NC
E_PER_TILE = E_PAD // NW

GK = 64
G_CHUNKS = E_PER_TILE // GK

C_CHUNKS = 25
EC_PAD = NW * C_CHUNKS * K
EC_PER_TILE = EC_PAD // NW

ROWS_PER_TILE = N_PAD // NS
HW = 16

_mesh = plsc.VectorSubcoreMesh(core_axis_name="c", subcore_axis_name="s")
_sc_params = pltpu.CompilerParams(needs_layout_passes=False)
_sc_params_flat = pltpu.CompilerParams(
    needs_layout_passes=False, use_tc_tiling_on_sc=False)


@functools.partial(
    pl.kernel,
    out_type=jax.ShapeDtypeStruct((NC, 2 * N_PAD, HW), jnp.float32),
    mesh=_mesh,
    compiler_params=_sc_params_flat,
    scratch_types=[
        pltpu.VMEM((G_CHUNKS, GK), jnp.int32),
        pltpu.VMEM((G_CHUNKS, GK), jnp.int32),
        pltpu.VMEM((GK, HW), jnp.float32),
        pltpu.VMEM_SHARED((2 * N_PAD, HW), jnp.float32),
        pltpu.SemaphoreType.DMA,
        pltpu.SemaphoreType.DMA,
        pltpu.SemaphoreType.DMA,
    ],
)
def _degrees(src_hbm, dst_hbm, out_hbm, sidx, didx, ones_v, hist_sh,
             isem, ssem, dsem):
    c = lax.axis_index("c")
    s = lax.axis_index("s")
    w = s * NC + c
    zero16 = jnp.zeros((HW,), jnp.float32)
    e0 = jnp.where(lax.iota(jnp.int32, HW) == 0, 1.0, 0.0).astype(jnp.float32)

    icp_s = pltpu.async_copy(src_hbm.at[w], sidx, isem)
    icp_d = pltpu.async_copy(dst_hbm.at[w], didx, isem)

    def fill(val):
        def body(i, _):
            ones_v[i, :] = val
            return 0
        lax.fori_loop(0, GK, body, 0)

    fill(zero16)
    hrows = (2 * N_PAD) // NS
    for t in range(hrows // GK):
        pltpu.sync_copy(ones_v, hist_sh.at[pl.ds(s * hrows + t * GK, GK)])
    fill(e0)
    icp_s.wait()
    icp_d.wait()

    def shift(k, _):
        for j in range(GK // 16):
            didx[k, pl.ds(j * 16, 16)] = didx[k, pl.ds(j * 16, 16)] + N_PAD
        return 0

    lax.fori_loop(0, G_CHUNKS, shift, 0)
    plsc.subcore_barrier()

    def step(k, _):
        cp_s = pltpu.async_copy(ones_v, hist_sh.at[sidx.at[k]], ssem, add=True)
        cp_d = pltpu.async_copy(ones_v, hist_sh.at[didx.at[k]], dsem, add=True)
        cp_s.wait()
        cp_d.wait()
        return 0

    lax.fori_loop(0, G_CHUNKS, step, 0)
    plsc.subcore_barrier()
    pltpu.sync_copy(hist_sh.at[pl.ds(s * hrows, hrows)],
                    out_hbm.at[c, pl.ds(s * hrows, hrows)])


@functools.partial(
    pl.kernel,
    out_type=jax.ShapeDtypeStruct((NC, N_PAD, D), jnp.float32),
    mesh=_mesh,
    compiler_params=_sc_params_flat,
    scratch_types=[
        pltpu.VMEM((G_CHUNKS, GK), jnp.int32),
        pltpu.VMEM((G_CHUNKS, GK), jnp.int32),
        pltpu.VMEM((2, GK, D), jnp.float32),
        pltpu.VMEM_SHARED((N_PAD, D), jnp.float32),
        pltpu.SemaphoreType.DMA,
        pltpu.SemaphoreType.DMA,
        pltpu.SemaphoreType.DMA,
    ],
)
def _aggregate(x_hbm, src_hbm, dst_hbm, out_hbm, sidx, didx, rows, acc_sh,
               isem, gsem0, gsem1):
    c = lax.axis_index("c")
    s = lax.axis_index("s")
    w = s * NC + c
    zerov = jnp.zeros((16,), jnp.float32)

    icp_s = pltpu.async_copy(src_hbm.at[w], sidx, isem)
    icp_d = pltpu.async_copy(dst_hbm.at[w], didx, isem)

    def zrow(i, _):
        for j in range(D // 16):
            rows[0, i, pl.ds(j * 16, 16)] = zerov
        return 0

    lax.fori_loop(0, GK, zrow, 0)
    base_r = s * ROWS_PER_TILE
    for t in range(ROWS_PER_TILE // GK):
        pltpu.sync_copy(rows.at[0], acc_sh.at[pl.ds(base_r + t * GK, GK)])
    icp_s.wait()
    icp_d.wait()
    plsc.subcore_barrier()

    pltpu.async_copy(x_hbm.at[sidx.at[0]], rows.at[0], gsem0)

    def pair(k2, _):
        k = 2 * k2

        @pl.when(k + 1 < G_CHUNKS)
        def _():
            pltpu.async_copy(x_hbm.at[sidx.at[k + 1]], rows.at[1], gsem1)

        pltpu.make_async_copy(x_hbm.at[sidx.at[0]], rows.at[0], gsem0).wait()
        pltpu.sync_copy(rows.at[0], acc_sh.at[didx.at[k]], add=True)

        @pl.when(k + 2 < G_CHUNKS)
        def _():
            pltpu.async_copy(x_hbm.at[sidx.at[k + 2]], rows.at[0], gsem0)

        @pl.when(k + 1 < G_CHUNKS)
        def _():
            pltpu.make_async_copy(x_hbm.at[sidx.at[0]], rows.at[1], gsem1).wait()
            pltpu.sync_copy(rows.at[1], acc_sh.at[didx.at[k + 1]], add=True)

        return 0

    lax.fori_loop(0, (G_CHUNKS + 1) // 2, pair, 0)
    plsc.subcore_barrier()
    pltpu.sync_copy(acc_sh.at[pl.ds(base_r, ROWS_PER_TILE)],
                    out_hbm.at[c, pl.ds(base_r, ROWS_PER_TILE)])


@functools.partial(
    pl.kernel,
    out_type=jax.ShapeDtypeStruct((EC_PAD * OUT_DIM,), jnp.float32),
    mesh=_mesh,
    compiler_params=_sc_params,
    scratch_types=[
        pltpu.VMEM((N_PAD * 8,), jnp.float32),
        pltpu.VMEM((C_CHUNKS, K), jnp.int32),
        pltpu.VMEM((C_CHUNKS, K), jnp.int32),
        pltpu.VMEM((K * OUT_DIM,), jnp.float32),
        pltpu.SemaphoreType.DMA,
    ],
)
def _classify(pq_hbm, s_hbm, d_hbm, out_hbm, pq_v, sidx, didx, stage, isem):
    c = lax.axis_index("c")
    s = lax.axis_index("s")
    w = s * NC + c
    icp_s = pltpu.async_copy(s_hbm.at[w], sidx, isem)
    icp_d = pltpu.async_copy(d_hbm.at[w], didx, isem)
    pltpu.sync_copy(pq_hbm, pq_v)
    icp_s.wait()
    icp_d.wait()
    lane = lax.iota(jnp.int32, 16)

    def step(k, _):
        base = w * EC_PER_TILE + k * K
        for j in range(K // 16):
            sv = sidx[k, pl.ds(j * 16, 16)] * 8
            dv = didx[k, pl.ds(j * 16, 16)] * 8
            p0 = plsc.load_gather(pq_v, [sv])
            p1 = plsc.load_gather(pq_v, [sv + 1])
            q0 = plsc.load_gather(pq_v, [dv + 2])
            q1 = plsc.load_gather(pq_v, [dv + 3])
            l0 = p0 + q0
            l1 = p1 + q1
            s0 = 1.0 / (1.0 + jnp.exp(-l0))
            s1 = 1.0 / (1.0 + jnp.exp(-l1))
            pos = j * 32 + lane * 2
            plsc.store_scatter(stage, [pos], s0)
            plsc.store_scatter(stage, [pos + 1], s1)
        pltpu.sync_copy(stage, out_hbm.at[pl.ds(base * OUT_DIM, K * OUT_DIM)])
        return 0

    lax.fori_loop(0, C_CHUNKS, step, 0)


BLK = 256
NB = N_PAD // BLK


def _prep_body(hist_ref, feat_ref, x1_ref):
    blk = hist_ref[...]
    deg = blk[0] + blk[1]
    ns = lax.rsqrt(jnp.maximum(deg[:, 0:1], 1.0))
    x1_ref[...] = feat_ref[...] * ns


_prep = pl.pallas_call(
    _prep_body,
    grid=(NB,),
    in_specs=[
        pl.BlockSpec((NC, BLK, HW), lambda i: (0, i, 0)),
        pl.BlockSpec((BLK, D), lambda i: (i, 0)),
    ],
    out_specs=pl.BlockSpec((BLK, D), lambda i: (i, 0)),
    out_shape=jax.ShapeDtypeStruct((N_PAD, D), jnp.float32),
)


def _layer1_body(agg_ref, hd_ref, hs_ref, w_ref, b_ref, out_ref):
    a = agg_ref[0] + agg_ref[1]
    dd = hd_ref[0] + hd_ref[1]
    nd = lax.rsqrt(jnp.maximum(dd[:, 0:1], 1.0))
    y = jnp.dot(a * nd, w_ref[...], preferred_element_type=jnp.float32)
    y = jnp.maximum(y + b_ref[...], 0.0)
    sd = hs_ref[0] + hs_ref[1]
    ns = lax.rsqrt(jnp.maximum(sd[:, 0:1], 1.0))
    out_ref[...] = y * ns


_layer1 = pl.pallas_call(
    _layer1_body,
    grid=(NB,),
    in_specs=[
        pl.BlockSpec((NC, BLK, D), lambda i: (0, i, 0)),
        pl.BlockSpec((NC, BLK, HW), lambda i: (0, NB + i, 0)),
        pl.BlockSpec((NC, BLK, HW), lambda i: (0, i, 0)),
        pl.BlockSpec((D, D), lambda i: (0, 0)),
        pl.BlockSpec((1, D), lambda i: (0, 0)),
    ],
    out_specs=pl.BlockSpec((BLK, D), lambda i: (i, 0)),
    out_shape=jax.ShapeDtypeStruct((N_PAD, D), jnp.float32),
)


def _layer2_body(agg_ref, hd_ref, w_ref, b_ref, wc_ref, bc_ref, h_ref, pq_ref):
    a = agg_ref[0] + agg_ref[1]
    dd = hd_ref[0] + hd_ref[1]
    nd = lax.rsqrt(jnp.maximum(dd[:, 0:1], 1.0))
    y = jnp.dot(a * nd, w_ref[...], preferred_element_type=jnp.float32)
    y = jnp.maximum(y + b_ref[...], 0.0)
    h_ref[...] = y
    pq_ref[...] = (
        jnp.dot(y, wc_ref[...], preferred_element_type=jnp.float32) + bc_ref[...]
    )


_layer2 = pl.pallas_call(
    _layer2_body,
    grid=(NB,),
    in_specs=[
        pl.BlockSpec((NC, BLK, D), lambda i: (0, i, 0)),
        pl.BlockSpec((NC, BLK, HW), lambda i: (0, NB + i, 0)),
        pl.BlockSpec((D, D), lambda i: (0, 0)),
        pl.BlockSpec((1, D), lambda i: (0, 0)),
        pl.BlockSpec((D, 8), lambda i: (0, 0)),
        pl.BlockSpec((1, 8), lambda i: (0, 0)),
    ],
    out_specs=[
        pl.BlockSpec((BLK, D), lambda i: (i, 0)),
        pl.BlockSpec((BLK, 8), lambda i: (i, 0)),
    ],
    out_shape=[
        jax.ShapeDtypeStruct((N_PAD, D), jnp.float32),
        jax.ShapeDtypeStruct((N_PAD, 8), jnp.float32),
    ],
)


def kernel(feat, graph_edge_index, edge_index, W1, b1, W2, b2, Wc, bc):
    f32 = jnp.float32
    feat_pad = jnp.pad(feat.astype(f32), ((0, N_PAD - N_NODES), (0, 0)))

    n_pad_e = E_PAD - N_EDGES
    pad_idx = (jnp.arange(n_pad_e, dtype=jnp.int32) % (N_PAD - N_NODES)) + N_NODES
    src_p = jnp.concatenate([graph_edge_index[0].astype(jnp.int32), pad_idx])
    dst_p = jnp.concatenate([graph_edge_index[1].astype(jnp.int32), pad_idx])
    src3 = src_p.reshape(NW, G_CHUNKS, GK)
    dst3 = dst_p.reshape(NW, G_CHUNKS, GK)

    n_pad_c = EC_PAD - N_CLS
    zpad = jnp.zeros((n_pad_c,), jnp.int32)
    cs3 = jnp.concatenate([edge_index[0].astype(jnp.int32), zpad]).reshape(
        NW, C_CHUNKS, K)
    cd3 = jnp.concatenate([edge_index[1].astype(jnp.int32), zpad]).reshape(
        NW, C_CHUNKS, K)

    b1r = b1.astype(f32).reshape(1, D)
    b2r = b2.astype(f32).reshape(1, D)
    wc8 = jnp.zeros((D, 8), f32)
    wc8 = wc8.at[:, 0:2].set(Wc[:D].astype(f32))
    wc8 = wc8.at[:, 2:4].set(Wc[D:].astype(f32))
    bc8 = jnp.zeros((1, 8), f32).at[0, 0:2].set(bc.astype(f32))

    hist = _degrees(src3, dst3)
    x1 = _prep(hist, feat_pad)
    agg1 = _aggregate(x1, src3, dst3)
    x2 = _layer1(agg1, hist, hist, W1.astype(f32), b1r)
    agg2 = _aggregate(x2, src3, dst3)
    h, pq = _layer2(agg2, hist, W2.astype(f32), b2r, wc8, bc8)
    probs = _classify(pq.reshape(N_PAD * 8), cs3, cd3)

    return h[:N_NODES], probs.reshape(EC_PAD, OUT_DIM)[:N_CLS]


# speedup vs baseline: 10.2619x; 1.3530x over previous
import functools

import jax
import jax.numpy as jnp
from jax import lax
from jax.experimental import pallas as pl
from jax.experimental.pallas import tpu as pltpu
from jax.experimental.pallas import tpu_sc as plsc

N_NODES = 10000
N_EDGES = 320000
N_CLS = 100000
D = 128
OUT_DIM = 2

N_PAD = 10240
K = 128
NC, NS = 2, 16
NW = NC * NS

E_CHUNKS = 79
E_PAD = NW * E_CHUNKS * K
E_PER_CORE = E_PAD // NC
E_PER_TILE = E_PAD // NW

GK = 64
G_CHUNKS = E_PER_TILE // GK

C_CHUNKS = 25
EC_PAD = NW * C_CHUNKS * K
EC_PER_TILE = EC_PAD // NW

ROWS_PER_TILE = N_PAD // NS
HW = 16

_mesh = plsc.VectorSubcoreMesh(core_axis_name="c", subcore_axis_name="s")
_sc_params = pltpu.CompilerParams(needs_layout_passes=False)
_sc_params_flat = pltpu.CompilerParams(
    needs_layout_passes=False, use_tc_tiling_on_sc=False)


@functools.partial(
    pl.kernel,
    out_type=jax.ShapeDtypeStruct((NC, 2 * N_PAD, HW), jnp.float32),
    mesh=_mesh,
    compiler_params=_sc_params_flat,
    scratch_types=[
        pltpu.VMEM((G_CHUNKS, GK), jnp.int32),
        pltpu.VMEM((G_CHUNKS, GK), jnp.int32),
        pltpu.VMEM((GK, HW), jnp.float32),
        pltpu.VMEM_SHARED((2 * N_PAD, HW), jnp.float32),
        pltpu.SemaphoreType.DMA,
        pltpu.SemaphoreType.DMA,
        pltpu.SemaphoreType.DMA,
    ],
)
def _degrees(src_hbm, dst_hbm, out_hbm, sidx, didx, ones_v, hist_sh,
             isem, ssem, dsem):
    c = lax.axis_index("c")
    s = lax.axis_index("s")
    w = s * NC + c
    zero16 = jnp.zeros((HW,), jnp.float32)
    e0 = jnp.where(lax.iota(jnp.int32, HW) == 0, 1.0, 0.0).astype(jnp.float32)

    icp_s = pltpu.async_copy(src_hbm.at[w], sidx, isem)
    icp_d = pltpu.async_copy(dst_hbm.at[w], didx, isem)

    def fill(val):
        def body(i, _):
            ones_v[i, :] = val
            return 0
        lax.fori_loop(0, GK, body, 0)

    fill(zero16)
    hrows = (2 * N_PAD) // NS
    for t in range(hrows // GK):
        pltpu.sync_copy(ones_v, hist_sh.at[pl.ds(s * hrows + t * GK, GK)])
    fill(e0)
    icp_s.wait()
    icp_d.wait()

    def shift(k, _):
        for j in range(GK // 16):
            didx[k, pl.ds(j * 16, 16)] = didx[k, pl.ds(j * 16, 16)] + N_PAD
        return 0

    lax.fori_loop(0, G_CHUNKS, shift, 0)
    plsc.subcore_barrier()

    def step(k, _):
        cp_s = pltpu.async_copy(ones_v, hist_sh.at[sidx.at[k]], ssem, add=True)
        cp_d = pltpu.async_copy(ones_v, hist_sh.at[didx.at[k]], dsem, add=True)
        cp_s.wait()
        cp_d.wait()
        return 0

    lax.fori_loop(0, G_CHUNKS, step, 0)
    plsc.subcore_barrier()
    pltpu.sync_copy(hist_sh.at[pl.ds(s * hrows, hrows)],
                    out_hbm.at[c, pl.ds(s * hrows, hrows)])


@functools.partial(
    pl.kernel,
    out_type=jax.ShapeDtypeStruct((NC, N_PAD, D), jnp.float32),
    mesh=_mesh,
    compiler_params=_sc_params_flat,
    scratch_types=[
        pltpu.VMEM((G_CHUNKS, GK), jnp.int32),
        pltpu.VMEM((G_CHUNKS, GK), jnp.int32),
        pltpu.VMEM((2, GK, D), jnp.float32),
        pltpu.VMEM_SHARED((N_PAD, D), jnp.float32),
        pltpu.SemaphoreType.DMA,
        pltpu.SemaphoreType.DMA,
        pltpu.SemaphoreType.DMA,
    ],
)
def _aggregate(x_hbm, src_hbm, dst_hbm, out_hbm, sidx, didx, rows, acc_sh,
               isem, gsem0, gsem1):
    c = lax.axis_index("c")
    s = lax.axis_index("s")
    w = s * NC + c
    zerov = jnp.zeros((16,), jnp.float32)

    icp_s = pltpu.async_copy(src_hbm.at[w], sidx, isem)
    icp_d = pltpu.async_copy(dst_hbm.at[w], didx, isem)

    def zrow(i, _):
        for j in range(D // 16):
            rows[0, i, pl.ds(j * 16, 16)] = zerov
        return 0

    lax.fori_loop(0, GK, zrow, 0)
    base_r = s * ROWS_PER_TILE
    for t in range(ROWS_PER_TILE // GK):
        pltpu.sync_copy(rows.at[0], acc_sh.at[pl.ds(base_r + t * GK, GK)])
    icp_s.wait()
    icp_d.wait()
    plsc.subcore_barrier()

    pltpu.async_copy(x_hbm.at[sidx.at[0]], rows.at[0], gsem0)

    def pair(k2, _):
        k = 2 * k2

        @pl.when(k + 1 < G_CHUNKS)
        def _():
            pltpu.async_copy(x_hbm.at[sidx.at[k + 1]], rows.at[1], gsem1)

        pltpu.make_async_copy(x_hbm.at[sidx.at[0]], rows.at[0], gsem0).wait()
        pltpu.sync_copy(rows.at[0], acc_sh.at[didx.at[k]], add=True)

        @pl.when(k + 2 < G_CHUNKS)
        def _():
            pltpu.async_copy(x_hbm.at[sidx.at[k + 2]], rows.at[0], gsem0)

        @pl.when(k + 1 < G_CHUNKS)
        def _():
            pltpu.make_async_copy(x_hbm.at[sidx.at[0]], rows.at[1], gsem1).wait()
            pltpu.sync_copy(rows.at[1], acc_sh.at[didx.at[k + 1]], add=True)

        return 0

    lax.fori_loop(0, (G_CHUNKS + 1) // 2, pair, 0)
    plsc.subcore_barrier()
    pltpu.sync_copy(acc_sh.at[pl.ds(base_r, ROWS_PER_TILE)],
                    out_hbm.at[c, pl.ds(base_r, ROWS_PER_TILE)])


@functools.partial(
    pl.kernel,
    out_type=jax.ShapeDtypeStruct((OUT_DIM, EC_PAD), jnp.float32),
    mesh=_mesh,
    compiler_params=_sc_params,
    scratch_types=[
        pltpu.VMEM((N_PAD * 8,), jnp.float32),
        pltpu.VMEM((C_CHUNKS, K), jnp.int32),
        pltpu.VMEM((C_CHUNKS, K), jnp.int32),
        pltpu.VMEM((K,), jnp.float32),
        pltpu.VMEM((K,), jnp.float32),
        pltpu.SemaphoreType.DMA,
    ],
)
def _classify(pq_hbm, s_hbm, d_hbm, out_hbm, pq_v, sidx, didx, st0, st1, isem):
    c = lax.axis_index("c")
    s = lax.axis_index("s")
    w = s * NC + c
    icp_s = pltpu.async_copy(s_hbm.at[w], sidx, isem)
    icp_d = pltpu.async_copy(d_hbm.at[w], didx, isem)
    pltpu.sync_copy(pq_hbm, pq_v)
    icp_s.wait()
    icp_d.wait()

    def step(k, _):
        base = w * EC_PER_TILE + k * K
        for j in range(K // 16):
            sv = sidx[k, pl.ds(j * 16, 16)] * 8
            dv = didx[k, pl.ds(j * 16, 16)] * 8
            p0 = plsc.load_gather(pq_v, [sv])
            p1 = plsc.load_gather(pq_v, [sv + 1])
            q0 = plsc.load_gather(pq_v, [dv + 2])
            q1 = plsc.load_gather(pq_v, [dv + 3])
            l0 = p0 + q0
            l1 = p1 + q1
            st0[pl.ds(j * 16, 16)] = 1.0 / (1.0 + jnp.exp(-l0))
            st1[pl.ds(j * 16, 16)] = 1.0 / (1.0 + jnp.exp(-l1))
        pltpu.sync_copy(st0, out_hbm.at[0, pl.ds(base, K)])
        pltpu.sync_copy(st1, out_hbm.at[1, pl.ds(base, K)])
        return 0

    lax.fori_loop(0, C_CHUNKS, step, 0)


BLK = 1024
NB = N_PAD // BLK


def _prep_body(hist_ref, feat_ref, x1_ref):
    blk = hist_ref[...]
    deg = blk[0] + blk[1]
    ns = lax.rsqrt(jnp.maximum(deg[:, 0:1], 1.0))
    x1_ref[...] = feat_ref[...] * ns


_prep = pl.pallas_call(
    _prep_body,
    grid=(NB,),
    in_specs=[
        pl.BlockSpec((NC, BLK, HW), lambda i: (0, i, 0)),
        pl.BlockSpec((BLK, D), lambda i: (i, 0)),
    ],
    out_specs=pl.BlockSpec((BLK, D), lambda i: (i, 0)),
    out_shape=jax.ShapeDtypeStruct((N_PAD, D), jnp.float32),
)


def _layer1_body(agg_ref, hd_ref, hs_ref, w_ref, b_ref, out_ref):
    a = agg_ref[0] + agg_ref[1]
    dd = hd_ref[0] + hd_ref[1]
    nd = lax.rsqrt(jnp.maximum(dd[:, 0:1], 1.0))
    y = jnp.dot(a * nd, w_ref[...], preferred_element_type=jnp.float32)
    y = jnp.maximum(y + b_ref[...], 0.0)
    sd = hs_ref[0] + hs_ref[1]
    ns = lax.rsqrt(jnp.maximum(sd[:, 0:1], 1.0))
    out_ref[...] = y * ns


_layer1 = pl.pallas_call(
    _layer1_body,
    grid=(NB,),
    in_specs=[
        pl.BlockSpec((NC, BLK, D), lambda i: (0, i, 0)),
        pl.BlockSpec((NC, BLK, HW), lambda i: (0, NB + i, 0)),
        pl.BlockSpec((NC, BLK, HW), lambda i: (0, i, 0)),
        pl.BlockSpec((D, D), lambda i: (0, 0)),
        pl.BlockSpec((1, D), lambda i: (0, 0)),
    ],
    out_specs=pl.BlockSpec((BLK, D), lambda i: (i, 0)),
    out_shape=jax.ShapeDtypeStruct((N_PAD, D), jnp.float32),
)


def _layer2_body(agg_ref, hd_ref, w_ref, b_ref, wc_ref, bc_ref, h_ref, pq_ref):
    a = agg_ref[0] + agg_ref[1]
    dd = hd_ref[0] + hd_ref[1]
    nd = lax.rsqrt(jnp.maximum(dd[:, 0:1], 1.0))
    y = jnp.dot(a * nd, w_ref[...], preferred_element_type=jnp.float32)
    y = jnp.maximum(y + b_ref[...], 0.0)
    h_ref[...] = y
    pq_ref[...] = (
        jnp.dot(y, wc_ref[...], preferred_element_type=jnp.float32) + bc_ref[...]
    )


_layer2 = pl.pallas_call(
    _layer2_body,
    grid=(NB,),
    in_specs=[
        pl.BlockSpec((NC, BLK, D), lambda i: (0, i, 0)),
        pl.BlockSpec((NC, BLK, HW), lambda i: (0, NB + i, 0)),
        pl.BlockSpec((D, D), lambda i: (0, 0)),
        pl.BlockSpec((1, D), lambda i: (0, 0)),
        pl.BlockSpec((D, 8), lambda i: (0, 0)),
        pl.BlockSpec((1, 8), lambda i: (0, 0)),
    ],
    out_specs=[
        pl.BlockSpec((BLK, D), lambda i: (i, 0)),
        pl.BlockSpec((BLK, 8), lambda i: (i, 0)),
    ],
    out_shape=[
        jax.ShapeDtypeStruct((N_PAD, D), jnp.float32),
        jax.ShapeDtypeStruct((N_PAD, 8), jnp.float32),
    ],
)


def kernel(feat, graph_edge_index, edge_index, W1, b1, W2, b2, Wc, bc):
    f32 = jnp.float32
    feat_pad = jnp.pad(feat.astype(f32), ((0, N_PAD - N_NODES), (0, 0)))

    n_pad_e = E_PAD - N_EDGES
    pad_idx = (jnp.arange(n_pad_e, dtype=jnp.int32) % (N_PAD - N_NODES)) + N_NODES
    src_p = jnp.concatenate([graph_edge_index[0].astype(jnp.int32), pad_idx])
    dst_p = jnp.concatenate([graph_edge_index[1].astype(jnp.int32), pad_idx])
    src3 = src_p.reshape(NW, G_CHUNKS, GK)
    dst3 = dst_p.reshape(NW, G_CHUNKS, GK)

    n_pad_c = EC_PAD - N_CLS
    zpad = jnp.zeros((n_pad_c,), jnp.int32)
    cs3 = jnp.concatenate([edge_index[0].astype(jnp.int32), zpad]).reshape(
        NW, C_CHUNKS, K)
    cd3 = jnp.concatenate([edge_index[1].astype(jnp.int32), zpad]).reshape(
        NW, C_CHUNKS, K)

    b1r = b1.astype(f32).reshape(1, D)
    b2r = b2.astype(f32).reshape(1, D)
    wc8 = jnp.zeros((D, 8), f32)
    wc8 = wc8.at[:, 0:2].set(Wc[:D].astype(f32))
    wc8 = wc8.at[:, 2:4].set(Wc[D:].astype(f32))
    bc8 = jnp.zeros((1, 8), f32).at[0, 0:2].set(bc.astype(f32))

    hist = _degrees(src3, dst3)
    x1 = _prep(hist, feat_pad)
    agg1 = _aggregate(x1, src3, dst3)
    x2 = _layer1(agg1, hist, hist, W1.astype(f32), b1r)
    agg2 = _aggregate(x2, src3, dst3)
    h, pq = _layer2(agg2, hist, W2.astype(f32), b2r, wc8, bc8)
    planes = _classify(pq.reshape(N_PAD * 8), cs3, cd3)

    probs = jnp.stack([planes[0, :N_CLS], planes[1, :N_CLS]], axis=1)
    return h[:N_NODES], probs


# speedup vs baseline: 12.1359x; 1.1826x over previous
import functools

import jax
import jax.numpy as jnp
from jax import lax
from jax.experimental import pallas as pl
from jax.experimental.pallas import tpu as pltpu
from jax.experimental.pallas import tpu_sc as plsc

N_NODES = 10000
N_EDGES = 320000
N_CLS = 100000
D = 128
OUT_DIM = 2

N_PAD = 10240
K = 128
NC, NS = 2, 16
NW = NC * NS

E_CHUNKS = 79
E_PAD = NW * E_CHUNKS * K
E_PER_CORE = E_PAD // NC
E_PER_TILE = E_PAD // NW

GK = 64
G_CHUNKS = E_PER_TILE // GK

C_CHUNKS = 25
EC_PAD = NW * C_CHUNKS * K
EC_PER_TILE = EC_PAD // NW

ROWS_PER_TILE = N_PAD // NS
HW = 16

_mesh = plsc.VectorSubcoreMesh(core_axis_name="c", subcore_axis_name="s")
_sc_params = pltpu.CompilerParams(needs_layout_passes=False)
_sc_params_flat = pltpu.CompilerParams(
    needs_layout_passes=False, use_tc_tiling_on_sc=False)


@functools.partial(
    pl.kernel,
    out_type=jax.ShapeDtypeStruct((NC, 2 * N_PAD, HW), jnp.float32),
    mesh=_mesh,
    compiler_params=_sc_params_flat,
    scratch_types=[
        pltpu.VMEM((G_CHUNKS, GK), jnp.int32),
        pltpu.VMEM((G_CHUNKS, GK), jnp.int32),
        pltpu.VMEM((GK, HW), jnp.float32),
        pltpu.VMEM_SHARED((2 * N_PAD, HW), jnp.float32),
        pltpu.SemaphoreType.DMA,
        pltpu.SemaphoreType.DMA,
        pltpu.SemaphoreType.DMA,
    ],
)
def _degrees(src_hbm, dst_hbm, out_hbm, sidx, didx, ones_v, hist_sh,
             isem, ssem, dsem):
    c = lax.axis_index("c")
    s = lax.axis_index("s")
    w = s * NC + c
    zero16 = jnp.zeros((HW,), jnp.float32)
    e0 = jnp.where(lax.iota(jnp.int32, HW) == 0, 1.0, 0.0).astype(jnp.float32)

    icp_s = pltpu.async_copy(src_hbm.at[w], sidx, isem)
    icp_d = pltpu.async_copy(dst_hbm.at[w], didx, isem)

    def fill(val):
        def body(i, _):
            ones_v[i, :] = val
            return 0
        lax.fori_loop(0, GK, body, 0)

    fill(zero16)
    hrows = (2 * N_PAD) // NS
    for t in range(hrows // GK):
        pltpu.sync_copy(ones_v, hist_sh.at[pl.ds(s * hrows + t * GK, GK)])
    fill(e0)
    icp_s.wait()
    icp_d.wait()

    def shift(k, _):
        for j in range(GK // 16):
            didx[k, pl.ds(j * 16, 16)] = didx[k, pl.ds(j * 16, 16)] + N_PAD
        return 0

    lax.fori_loop(0, G_CHUNKS, shift, 0)
    plsc.subcore_barrier()

    def step(k, _):
        cp_s = pltpu.async_copy(ones_v, hist_sh.at[sidx.at[k]], ssem, add=True)
        cp_d = pltpu.async_copy(ones_v, hist_sh.at[didx.at[k]], dsem, add=True)
        cp_s.wait()
        cp_d.wait()
        return 0

    lax.fori_loop(0, G_CHUNKS, step, 0)
    plsc.subcore_barrier()
    pltpu.sync_copy(hist_sh.at[pl.ds(s * hrows, hrows)],
                    out_hbm.at[c, pl.ds(s * hrows, hrows)])


@functools.partial(
    pl.kernel,
    out_type=jax.ShapeDtypeStruct((NC, N_PAD, D), jnp.float32),
    mesh=_mesh,
    compiler_params=_sc_params_flat,
    scratch_types=[
        pltpu.VMEM((G_CHUNKS, GK), jnp.int32),
        pltpu.VMEM((G_CHUNKS, GK), jnp.int32),
        pltpu.VMEM((3, GK, D), jnp.float32),
        pltpu.VMEM_SHARED((N_PAD, D), jnp.float32),
        pltpu.SemaphoreType.DMA,
        (pltpu.SemaphoreType.DMA, pltpu.SemaphoreType.DMA,
         pltpu.SemaphoreType.DMA),
        (pltpu.SemaphoreType.DMA, pltpu.SemaphoreType.DMA,
         pltpu.SemaphoreType.DMA),
    ],
)
def _aggregate(x_hbm, src_hbm, dst_hbm, out_hbm, sidx, didx, rows, acc_sh,
               isem, gsems, ssems):
    c = lax.axis_index("c")
    s = lax.axis_index("s")
    w = s * NC + c
    zerov = jnp.zeros((16,), jnp.float32)

    icp_s = pltpu.async_copy(src_hbm.at[w], sidx, isem)
    icp_d = pltpu.async_copy(dst_hbm.at[w], didx, isem)

    def zrow(i, _):
        for j in range(D // 16):
            rows[0, i, pl.ds(j * 16, 16)] = zerov
        return 0

    lax.fori_loop(0, GK, zrow, 0)
    base_r = s * ROWS_PER_TILE
    for t in range(ROWS_PER_TILE // GK):
        pltpu.sync_copy(rows.at[0], acc_sh.at[pl.ds(base_r + t * GK, GK)])
    icp_s.wait()
    icp_d.wait()
    plsc.subcore_barrier()

    def gather(k, b):
        pltpu.async_copy(x_hbm.at[sidx.at[k]], rows.at[b], gsems[b])

    def wait_gather(b):
        pltpu.make_async_copy(x_hbm.at[sidx.at[0]], rows.at[b], gsems[b]).wait()

    def scatter(k, b):
        pltpu.async_copy(rows.at[b], acc_sh.at[didx.at[k]], ssems[b], add=True)

    def wait_scatter(b):
        pltpu.make_async_copy(rows.at[b], acc_sh.at[didx.at[0]],
                              ssems[b]).wait()

    gather(0, 0)
    gather(1, 1)

    def tri(k3, _):
        for j in range(3):
            kk = 3 * k3 + j
            wait_gather(j)
            scatter(kk, j)
            nb = (j + 2) % 3

            @pl.when(kk >= 1)
            def _():
                wait_scatter(nb)

            gather(kk + 2, nb)
        return 0

    lax.fori_loop(0, (G_CHUNKS - 2) // 3, tri, 0)
    for kk in range(G_CHUNKS - 2, G_CHUNKS):
        b = kk % 3
        wait_gather(b)
        scatter(kk, b)
    for b in range(3):
        wait_scatter(b)
    plsc.subcore_barrier()
    pltpu.sync_copy(acc_sh.at[pl.ds(base_r, ROWS_PER_TILE)],
                    out_hbm.at[c, pl.ds(base_r, ROWS_PER_TILE)])


@functools.partial(
    pl.kernel,
    out_type=jax.ShapeDtypeStruct((OUT_DIM, EC_PAD), jnp.float32),
    mesh=_mesh,
    compiler_params=_sc_params,
    scratch_types=[
        pltpu.VMEM((N_PAD * 8,), jnp.float32),
        pltpu.VMEM((C_CHUNKS, K), jnp.int32),
        pltpu.VMEM((C_CHUNKS, K), jnp.int32),
        pltpu.VMEM((K,), jnp.float32),
        pltpu.VMEM((K,), jnp.float32),
        pltpu.SemaphoreType.DMA,
    ],
)
def _classify(pq_hbm, s_hbm, d_hbm, out_hbm, pq_v, sidx, didx, st0, st1, isem):
    c = lax.axis_index("c")
    s = lax.axis_index("s")
    w = s * NC + c
    icp_s = pltpu.async_copy(s_hbm.at[w], sidx, isem)
    icp_d = pltpu.async_copy(d_hbm.at[w], didx, isem)
    pltpu.sync_copy(pq_hbm, pq_v)
    icp_s.wait()
    icp_d.wait()

    def step(k, _):
        base = w * EC_PER_TILE + k * K
        for j in range(K // 16):
            sv = sidx[k, pl.ds(j * 16, 16)] * 8
            dv = didx[k, pl.ds(j * 16, 16)] * 8
            p0 = plsc.load_gather(pq_v, [sv])
            p1 = plsc.load_gather(pq_v, [sv + 1])
            q0 = plsc.load_gather(pq_v, [dv + 2])
            q1 = plsc.load_gather(pq_v, [dv + 3])
            l0 = p0 + q0
            l1 = p1 + q1
            st0[pl.ds(j * 16, 16)] = 1.0 / (1.0 + jnp.exp(-l0))
            st1[pl.ds(j * 16, 16)] = 1.0 / (1.0 + jnp.exp(-l1))
        pltpu.sync_copy(st0, out_hbm.at[0, pl.ds(base, K)])
        pltpu.sync_copy(st1, out_hbm.at[1, pl.ds(base, K)])
        return 0

    lax.fori_loop(0, C_CHUNKS, step, 0)


BLK = 1024
NB = N_PAD // BLK
HB = BLK * HW // 128


def _deg_col(blk):
    d = blk[0] + blk[1]
    d = d.reshape(HB, 8, 16)[:, :, 0]
    return d.reshape(BLK, 1)


def _prep_body(hist_ref, feat_ref, x1_ref):
    ns = lax.rsqrt(jnp.maximum(_deg_col(hist_ref[...]), 1.0))
    x1_ref[...] = feat_ref[...] * ns


_prep = pl.pallas_call(
    _prep_body,
    grid=(NB,),
    in_specs=[
        pl.BlockSpec((NC, HB, 128), lambda i: (0, i, 0)),
        pl.BlockSpec((BLK, D), lambda i: (i, 0)),
    ],
    out_specs=pl.BlockSpec((BLK, D), lambda i: (i, 0)),
    out_shape=jax.ShapeDtypeStruct((N_PAD, D), jnp.float32),
)


def _layer1_body(agg_ref, hd_ref, hs_ref, w_ref, b_ref, out_ref):
    a = agg_ref[0] + agg_ref[1]
    nd = lax.rsqrt(jnp.maximum(_deg_col(hd_ref[...]), 1.0))
    y = jnp.dot(a * nd, w_ref[...], preferred_element_type=jnp.float32)
    y = jnp.maximum(y + b_ref[...], 0.0)
    ns = lax.rsqrt(jnp.maximum(_deg_col(hs_ref[...]), 1.0))
    out_ref[...] = y * ns


_layer1 = pl.pallas_call(
    _layer1_body,
    grid=(NB,),
    in_specs=[
        pl.BlockSpec((NC, BLK, D), lambda i: (0, i, 0)),
        pl.BlockSpec((NC, HB, 128), lambda i: (0, NB + i, 0)),
        pl.BlockSpec((NC, HB, 128), lambda i: (0, i, 0)),
        pl.BlockSpec((D, D), lambda i: (0, 0)),
        pl.BlockSpec((1, D), lambda i: (0, 0)),
    ],
    out_specs=pl.BlockSpec((BLK, D), lambda i: (i, 0)),
    out_shape=jax.ShapeDtypeStruct((N_PAD, D), jnp.float32),
)


def _layer2_body(agg_ref, hd_ref, w_ref, b_ref, wc_ref, bc_ref, h_ref, pq_ref):
    a = agg_ref[0] + agg_ref[1]
    nd = lax.rsqrt(jnp.maximum(_deg_col(hd_ref[...]), 1.0))
    y = jnp.dot(a * nd, w_ref[...], preferred_element_type=jnp.float32)
    y = jnp.maximum(y + b_ref[...], 0.0)
    h_ref[...] = y
    pq_ref[...] = (
        jnp.dot(y, wc_ref[...], preferred_element_type=jnp.float32) + bc_ref[...]
    )


_layer2 = pl.pallas_call(
    _layer2_body,
    grid=(NB,),
    in_specs=[
        pl.BlockSpec((NC, BLK, D), lambda i: (0, i, 0)),
        pl.BlockSpec((NC, HB, 128), lambda i: (0, NB + i, 0)),
        pl.BlockSpec((D, D), lambda i: (0, 0)),
        pl.BlockSpec((1, D), lambda i: (0, 0)),
        pl.BlockSpec((D, 8), lambda i: (0, 0)),
        pl.BlockSpec((1, 8), lambda i: (0, 0)),
    ],
    out_specs=[
        pl.BlockSpec((BLK, D), lambda i: (i, 0)),
        pl.BlockSpec((BLK, 8), lambda i: (i, 0)),
    ],
    out_shape=[
        jax.ShapeDtypeStruct((N_PAD, D), jnp.float32),
        jax.ShapeDtypeStruct((N_PAD, 8), jnp.float32),
    ],
)


def kernel(feat, graph_edge_index, edge_index, W1, b1, W2, b2, Wc, bc):
    f32 = jnp.float32
    feat_pad = jnp.pad(feat.astype(f32), ((0, N_PAD - N_NODES), (0, 0)))

    n_pad_e = E_PAD - N_EDGES
    pad_idx = (jnp.arange(n_pad_e, dtype=jnp.int32) % (N_PAD - N_NODES)) + N_NODES
    src_p = jnp.concatenate([graph_edge_index[0].astype(jnp.int32), pad_idx])
    dst_p = jnp.concatenate([graph_edge_index[1].astype(jnp.int32), pad_idx])
    src3 = src_p.reshape(NW, G_CHUNKS, GK)
    dst3 = dst_p.reshape(NW, G_CHUNKS, GK)

    n_pad_c = EC_PAD - N_CLS
    zpad = jnp.zeros((n_pad_c,), jnp.int32)
    cs3 = jnp.concatenate([edge_index[0].astype(jnp.int32), zpad]).reshape(
        NW, C_CHUNKS, K)
    cd3 = jnp.concatenate([edge_index[1].astype(jnp.int32), zpad]).reshape(
        NW, C_CHUNKS, K)

    b1r = b1.astype(f32).reshape(1, D)
    b2r = b2.astype(f32).reshape(1, D)
    wc8 = jnp.zeros((D, 8), f32)
    wc8 = wc8.at[:, 0:2].set(Wc[:D].astype(f32))
    wc8 = wc8.at[:, 2:4].set(Wc[D:].astype(f32))
    bc8 = jnp.zeros((1, 8), f32).at[0, 0:2].set(bc.astype(f32))

    hist = _degrees(src3, dst3)
    hist2 = hist.reshape(NC, 2 * N_PAD * HW // 128, 128)
    x1 = _prep(hist2, feat_pad)
    agg1 = _aggregate(x1, src3, dst3)
    x2 = _layer1(agg1, hist2, hist2, W1.astype(f32), b1r)
    agg2 = _aggregate(x2, src3, dst3)
    h, pq = _layer2(agg2, hist2, W2.astype(f32), b2r, wc8, bc8)
    planes = _classify(pq.reshape(N_PAD * 8), cs3, cd3)

    probs = jnp.stack([planes[0, :N_CLS], planes[1, :N_CLS]], axis=1)
    return h[:N_NODES], probs


# speedup vs baseline: 12.2539x; 1.0097x over previous
import functools

import jax
import jax.numpy as jnp
from jax import lax
from jax.experimental import pallas as pl
from jax.experimental.pallas import tpu as pltpu
from jax.experimental.pallas import tpu_sc as plsc

N_NODES = 10000
N_EDGES = 320000
N_CLS = 100000
D = 128
OUT_DIM = 2

N_PAD = 10240
K = 128
NC, NS = 2, 16
NW = NC * NS

E_CHUNKS = 79
E_PAD = NW * E_CHUNKS * K
E_PER_CORE = E_PAD // NC
E_PER_TILE = E_PAD // NW

GK = 64
G_CHUNKS = E_PER_TILE // GK

C_CHUNKS = 25
EC_PAD = NW * C_CHUNKS * K
EC_PER_TILE = EC_PAD // NW

ROWS_PER_TILE = N_PAD // NS
HW = 16

_mesh = plsc.VectorSubcoreMesh(core_axis_name="c", subcore_axis_name="s")
_sc_params = pltpu.CompilerParams(needs_layout_passes=False)
_sc_params_flat = pltpu.CompilerParams(
    needs_layout_passes=False, use_tc_tiling_on_sc=False)


@functools.partial(
    pl.kernel,
    out_type=jax.ShapeDtypeStruct((NC, 2 * N_PAD, HW), jnp.float32),
    mesh=_mesh,
    compiler_params=_sc_params_flat,
    scratch_types=[
        pltpu.VMEM((G_CHUNKS, GK), jnp.int32),
        pltpu.VMEM((G_CHUNKS, GK), jnp.int32),
        pltpu.VMEM((GK, HW), jnp.float32),
        pltpu.VMEM_SHARED((2 * N_PAD, HW), jnp.float32),
        pltpu.SemaphoreType.DMA,
        pltpu.SemaphoreType.DMA,
        pltpu.SemaphoreType.DMA,
    ],
)
def _degrees(src_hbm, dst_hbm, out_hbm, sidx, didx, ones_v, hist_sh,
             isem, ssem, dsem):
    c = lax.axis_index("c")
    s = lax.axis_index("s")
    w = s * NC + c
    zero16 = jnp.zeros((HW,), jnp.float32)
    e0 = jnp.where(lax.iota(jnp.int32, HW) == 0, 1.0, 0.0).astype(jnp.float32)

    icp_s = pltpu.async_copy(src_hbm.at[w], sidx, isem)
    icp_d = pltpu.async_copy(dst_hbm.at[w], didx, isem)

    def fill(val):
        def body(i, _):
            ones_v[i, :] = val
            return 0
        lax.fori_loop(0, GK, body, 0)

    fill(zero16)
    hrows = (2 * N_PAD) // NS
    for t in range(hrows // GK):
        pltpu.sync_copy(ones_v, hist_sh.at[pl.ds(s * hrows + t * GK, GK)])
    fill(e0)
    icp_s.wait()
    icp_d.wait()

    def shift(k, _):
        for j in range(GK // 16):
            didx[k, pl.ds(j * 16, 16)] = didx[k, pl.ds(j * 16, 16)] + N_PAD
        return 0

    lax.fori_loop(0, G_CHUNKS, shift, 0)
    plsc.subcore_barrier()

    def step(k, _):
        cp_s = pltpu.async_copy(ones_v, hist_sh.at[sidx.at[k]], ssem, add=True)
        cp_d = pltpu.async_copy(ones_v, hist_sh.at[didx.at[k]], dsem, add=True)
        cp_s.wait()
        cp_d.wait()
        return 0

    lax.fori_loop(0, G_CHUNKS, step, 0)
    plsc.subcore_barrier()
    pltpu.sync_copy(hist_sh.at[pl.ds(s * hrows, hrows)],
                    out_hbm.at[c, pl.ds(s * hrows, hrows)])


@functools.partial(
    pl.kernel,
    out_type=jax.ShapeDtypeStruct((NC, N_PAD, D), jnp.float32),
    mesh=_mesh,
    compiler_params=_sc_params_flat,
    scratch_types=[
        pltpu.VMEM((G_CHUNKS, GK), jnp.int32),
        pltpu.VMEM((G_CHUNKS, GK), jnp.int32),
        pltpu.VMEM((3, GK, D), jnp.float32),
        pltpu.VMEM_SHARED((N_PAD, D), jnp.float32),
        pltpu.SemaphoreType.DMA,
        (pltpu.SemaphoreType.DMA, pltpu.SemaphoreType.DMA,
         pltpu.SemaphoreType.DMA),
        (pltpu.SemaphoreType.DMA, pltpu.SemaphoreType.DMA,
         pltpu.SemaphoreType.DMA),
    ],
)
def _aggregate(x_hbm, src_hbm, dst_hbm, out_hbm, sidx, didx, rows, acc_sh,
               isem, gsems, ssems):
    c = lax.axis_index("c")
    s = lax.axis_index("s")
    w = s * NC + c
    zerov = jnp.zeros((16,), jnp.float32)

    icp_s = pltpu.async_copy(src_hbm.at[w], sidx, isem)
    icp_d = pltpu.async_copy(dst_hbm.at[w], didx, isem)

    def zrow(i, _):
        for j in range(D // 16):
            rows[0, i, pl.ds(j * 16, 16)] = zerov
        return 0

    lax.fori_loop(0, GK, zrow, 0)
    base_r = s * ROWS_PER_TILE
    for t in range(ROWS_PER_TILE // GK):
        pltpu.sync_copy(rows.at[0], acc_sh.at[pl.ds(base_r + t * GK, GK)])
    icp_s.wait()
    icp_d.wait()
    plsc.subcore_barrier()

    def gather(k, b):
        pltpu.async_copy(x_hbm.at[sidx.at[k]], rows.at[b], gsems[b])

    def wait_gather(b):
        pltpu.make_async_copy(x_hbm.at[sidx.at[0]], rows.at[b], gsems[b]).wait()

    def scatter(k, b):
        pltpu.async_copy(rows.at[b], acc_sh.at[didx.at[k]], ssems[b], add=True)

    def wait_scatter(b):
        pltpu.make_async_copy(rows.at[b], acc_sh.at[didx.at[0]],
                              ssems[b]).wait()

    gather(0, 0)
    gather(1, 1)

    def tri(k3, _):
        for j in range(3):
            kk = 3 * k3 + j
            wait_gather(j)
            scatter(kk, j)
            nb = (j + 2) % 3

            @pl.when(kk >= 1)
            def _():
                wait_scatter(nb)

            gather(kk + 2, nb)
        return 0

    lax.fori_loop(0, (G_CHUNKS - 2) // 3, tri, 0)
    for kk in range(G_CHUNKS - 2, G_CHUNKS):
        b = kk % 3
        wait_gather(b)
        scatter(kk, b)
    for b in range(3):
        wait_scatter(b)
    plsc.subcore_barrier()
    pltpu.sync_copy(acc_sh.at[pl.ds(base_r, ROWS_PER_TILE)],
                    out_hbm.at[c, pl.ds(base_r, ROWS_PER_TILE)])


@functools.partial(
    pl.kernel,
    out_type=jax.ShapeDtypeStruct((OUT_DIM, EC_PAD), jnp.float32),
    mesh=_mesh,
    compiler_params=_sc_params,
    scratch_types=[
        pltpu.VMEM((N_PAD,), jnp.float32),
        pltpu.VMEM((N_PAD,), jnp.float32),
        pltpu.VMEM((N_PAD,), jnp.float32),
        pltpu.VMEM((N_PAD,), jnp.float32),
        pltpu.VMEM((16,), jnp.float32),
        pltpu.VMEM((C_CHUNKS, K), jnp.int32),
        pltpu.VMEM((C_CHUNKS, K), jnp.int32),
        pltpu.VMEM((K,), jnp.float32),
        pltpu.VMEM((K,), jnp.float32),
        pltpu.SemaphoreType.DMA,
    ],
)
def _classify(pq_hbm, bc_hbm, s_hbm, d_hbm, out_hbm,
              p0v, p1v, q0v, q1v, bcv, sidx, didx, st0, st1, isem):
    c = lax.axis_index("c")
    s = lax.axis_index("s")
    w = s * NC + c
    icp_s = pltpu.async_copy(s_hbm.at[w], sidx, isem)
    icp_d = pltpu.async_copy(d_hbm.at[w], didx, isem)
    pltpu.sync_copy(pq_hbm.at[0], p0v)
    pltpu.sync_copy(pq_hbm.at[1], p1v)
    pltpu.sync_copy(pq_hbm.at[2], q0v)
    pltpu.sync_copy(pq_hbm.at[3], q1v)
    pltpu.sync_copy(bc_hbm, bcv)
    zid = jnp.zeros((16,), jnp.int32)
    b0 = plsc.load_gather(bcv, [zid])
    b1 = plsc.load_gather(bcv, [zid + 1])
    icp_s.wait()
    icp_d.wait()

    def step(k, _):
        base = w * EC_PER_TILE + k * K
        for j in range(K // 16):
            sv = sidx[k, pl.ds(j * 16, 16)]
            dv = didx[k, pl.ds(j * 16, 16)]
            l0 = plsc.load_gather(p0v, [sv]) + plsc.load_gather(q0v, [dv]) + b0
            l1 = plsc.load_gather(p1v, [sv]) + plsc.load_gather(q1v, [dv]) + b1
            st0[pl.ds(j * 16, 16)] = 1.0 / (1.0 + jnp.exp(-l0))
            st1[pl.ds(j * 16, 16)] = 1.0 / (1.0 + jnp.exp(-l1))
        pltpu.sync_copy(st0, out_hbm.at[0, pl.ds(base, K)])
        pltpu.sync_copy(st1, out_hbm.at[1, pl.ds(base, K)])
        return 0

    lax.fori_loop(0, C_CHUNKS, step, 0)


BLK = 1024
NB = N_PAD // BLK
HB = BLK * HW // 128


def _deg_col(blk):
    d = blk[0] + blk[1]
    d = d.reshape(HB, 8, 16)[:, :, 0]
    return d.reshape(BLK, 1)


def _prep_body(hist_ref, feat_ref, x1_ref):
    ns = lax.rsqrt(jnp.maximum(_deg_col(hist_ref[...]), 1.0))
    x1_ref[...] = feat_ref[...] * ns


_prep = pl.pallas_call(
    _prep_body,
    grid=(NB,),
    in_specs=[
        pl.BlockSpec((NC, HB, 128), lambda i: (0, i, 0)),
        pl.BlockSpec((BLK, D), lambda i: (i, 0)),
    ],
    out_specs=pl.BlockSpec((BLK, D), lambda i: (i, 0)),
    out_shape=jax.ShapeDtypeStruct((N_PAD, D), jnp.float32),
)


def _layer1_body(agg_ref, hd_ref, hs_ref, w_ref, b_ref, out_ref):
    a = agg_ref[0] + agg_ref[1]
    nd = lax.rsqrt(jnp.maximum(_deg_col(hd_ref[...]), 1.0))
    y = jnp.dot(a * nd, w_ref[...], preferred_element_type=jnp.float32)
    y = jnp.maximum(y + b_ref[...], 0.0)
    ns = lax.rsqrt(jnp.maximum(_deg_col(hs_ref[...]), 1.0))
    out_ref[...] = y * ns


_layer1 = pl.pallas_call(
    _layer1_body,
    grid=(NB,),
    in_specs=[
        pl.BlockSpec((NC, BLK, D), lambda i: (0, i, 0)),
        pl.BlockSpec((NC, HB, 128), lambda i: (0, NB + i, 0)),
        pl.BlockSpec((NC, HB, 128), lambda i: (0, i, 0)),
        pl.BlockSpec((D, D), lambda i: (0, 0)),
        pl.BlockSpec((1, D), lambda i: (0, 0)),
    ],
    out_specs=pl.BlockSpec((BLK, D), lambda i: (i, 0)),
    out_shape=jax.ShapeDtypeStruct((N_PAD, D), jnp.float32),
)


def _layer2_body(agg_ref, hd_ref, w_ref, b_ref, wc_ref, h_ref, pq_ref):
    a = agg_ref[0] + agg_ref[1]
    nd = lax.rsqrt(jnp.maximum(_deg_col(hd_ref[...]), 1.0))
    y = jnp.dot(a * nd, w_ref[...], preferred_element_type=jnp.float32)
    y = jnp.maximum(y + b_ref[...], 0.0)
    h_ref[...] = y
    for j in range(4):
        pj = jnp.dot(y, wc_ref[...][:, j], preferred_element_type=jnp.float32)
        pq_ref[j] = pj.reshape(BLK // 128, 128)


_layer2 = pl.pallas_call(
    _layer2_body,
    grid=(NB,),
    in_specs=[
        pl.BlockSpec((NC, BLK, D), lambda i: (0, i, 0)),
        pl.BlockSpec((NC, HB, 128), lambda i: (0, NB + i, 0)),
        pl.BlockSpec((D, D), lambda i: (0, 0)),
        pl.BlockSpec((1, D), lambda i: (0, 0)),
        pl.BlockSpec((D, 8), lambda i: (0, 0)),
    ],
    out_specs=[
        pl.BlockSpec((BLK, D), lambda i: (i, 0)),
        pl.BlockSpec((4, BLK // 128, 128), lambda i: (0, i, 0)),
    ],
    out_shape=[
        jax.ShapeDtypeStruct((N_PAD, D), jnp.float32),
        jax.ShapeDtypeStruct((4, N_PAD // 128, 128), jnp.float32),
    ],
)


def kernel(feat, graph_edge_index, edge_index, W1, b1, W2, b2, Wc, bc):
    f32 = jnp.float32
    feat_pad = jnp.pad(feat.astype(f32), ((0, N_PAD - N_NODES), (0, 0)))

    n_pad_e = E_PAD - N_EDGES
    pad_idx = (jnp.arange(n_pad_e, dtype=jnp.int32) % (N_PAD - N_NODES)) + N_NODES
    src_p = jnp.concatenate([graph_edge_index[0].astype(jnp.int32), pad_idx])
    dst_p = jnp.concatenate([graph_edge_index[1].astype(jnp.int32), pad_idx])
    src3 = src_p.reshape(NW, G_CHUNKS, GK)
    dst3 = dst_p.reshape(NW, G_CHUNKS, GK)

    n_pad_c = EC_PAD - N_CLS
    zpad = jnp.zeros((n_pad_c,), jnp.int32)
    cs3 = jnp.concatenate([edge_index[0].astype(jnp.int32), zpad]).reshape(
        NW, C_CHUNKS, K)
    cd3 = jnp.concatenate([edge_index[1].astype(jnp.int32), zpad]).reshape(
        NW, C_CHUNKS, K)

    b1r = b1.astype(f32).reshape(1, D)
    b2r = b2.astype(f32).reshape(1, D)
    wc8 = jnp.zeros((D, 8), f32)
    wc8 = wc8.at[:, 0:2].set(Wc[:D].astype(f32))
    wc8 = wc8.at[:, 2:4].set(Wc[D:].astype(f32))
    bc16 = jnp.zeros((16,), f32).at[0:2].set(bc.astype(f32))

    hist = _degrees(src3, dst3)
    hist2 = hist.reshape(NC, 2 * N_PAD * HW // 128, 128)
    x1 = _prep(hist2, feat_pad)
    agg1 = _aggregate(x1, src3, dst3)
    x2 = _layer1(agg1, hist2, hist2, W1.astype(f32), b1r)
    agg2 = _aggregate(x2, src3, dst3)
    h, pq = _layer2(agg2, hist2, W2.astype(f32), b2r, wc8)
    planes = _classify(pq.reshape(4, N_PAD), bc16, cs3, cd3)

    probs = jnp.stack([planes[0, :N_CLS], planes[1, :N_CLS]], axis=1)
    return h[:N_NODES], probs


# speedup vs baseline: 12.5677x; 1.0256x over previous
import functools

import jax
import jax.numpy as jnp
from jax import lax
from jax.experimental import pallas as pl
from jax.experimental.pallas import tpu as pltpu
from jax.experimental.pallas import tpu_sc as plsc

N_NODES = 10000
N_EDGES = 320000
N_CLS = 100000
D = 128
OUT_DIM = 2

N_PAD = 10240
K = 128
NC, NS = 2, 16
NW = NC * NS

E_CHUNKS = 79
E_PAD = NW * E_CHUNKS * K
E_PER_CORE = E_PAD // NC
E_PER_TILE = E_PAD // NW

GK = 64
G_CHUNKS = E_PER_TILE // GK

C_CHUNKS = 25
EC_PAD = NW * C_CHUNKS * K
EC_PER_TILE = EC_PAD // NW

ROWS_PER_TILE = N_PAD // NS
HW = 16

_mesh = plsc.VectorSubcoreMesh(core_axis_name="c", subcore_axis_name="s")
_sc_params = pltpu.CompilerParams(needs_layout_passes=False)
_sc_params_flat = pltpu.CompilerParams(
    needs_layout_passes=False, use_tc_tiling_on_sc=False)


@functools.partial(
    pl.kernel,
    out_type=jax.ShapeDtypeStruct((NC, 2 * N_PAD, HW), jnp.float32),
    mesh=_mesh,
    compiler_params=_sc_params_flat,
    scratch_types=[
        pltpu.VMEM((G_CHUNKS, GK), jnp.int32),
        pltpu.VMEM((G_CHUNKS, GK), jnp.int32),
        pltpu.VMEM((GK, HW), jnp.float32),
        pltpu.VMEM_SHARED((2 * N_PAD, HW), jnp.float32),
        pltpu.SemaphoreType.DMA,
        pltpu.SemaphoreType.DMA,
        pltpu.SemaphoreType.DMA,
    ],
)
def _degrees(src_hbm, dst_hbm, out_hbm, sidx, didx, ones_v, hist_sh,
             isem, ssem, dsem):
    c = lax.axis_index("c")
    s = lax.axis_index("s")
    w = s * NC + c
    zero16 = jnp.zeros((HW,), jnp.float32)
    e0 = jnp.where(lax.iota(jnp.int32, HW) == 0, 1.0, 0.0).astype(jnp.float32)

    icp_s = pltpu.async_copy(src_hbm.at[w], sidx, isem)
    icp_d = pltpu.async_copy(dst_hbm.at[w], didx, isem)

    def fill(val):
        def body(i, _):
            ones_v[i, :] = val
            return 0
        lax.fori_loop(0, GK, body, 0)

    fill(zero16)
    hrows = (2 * N_PAD) // NS
    for t in range(hrows // GK):
        pltpu.sync_copy(ones_v, hist_sh.at[pl.ds(s * hrows + t * GK, GK)])
    fill(e0)
    icp_s.wait()
    icp_d.wait()

    def shift(k, _):
        for j in range(GK // 16):
            didx[k, pl.ds(j * 16, 16)] = didx[k, pl.ds(j * 16, 16)] + N_PAD
        return 0

    lax.fori_loop(0, G_CHUNKS, shift, 0)
    plsc.subcore_barrier()

    LAG = 4

    def wait_pair():
        pltpu.make_async_copy(ones_v, hist_sh.at[sidx.at[0]], ssem).wait()
        pltpu.make_async_copy(ones_v, hist_sh.at[didx.at[0]], dsem).wait()

    def step(k, _):
        pltpu.async_copy(ones_v, hist_sh.at[sidx.at[k]], ssem, add=True)
        pltpu.async_copy(ones_v, hist_sh.at[didx.at[k]], dsem, add=True)

        @pl.when(k >= LAG)
        def _():
            wait_pair()

        return 0

    lax.fori_loop(0, G_CHUNKS, step, 0)
    for _ in range(LAG):
        wait_pair()
    plsc.subcore_barrier()
    pltpu.sync_copy(hist_sh.at[pl.ds(s * hrows, hrows)],
                    out_hbm.at[c, pl.ds(s * hrows, hrows)])


@functools.partial(
    pl.kernel,
    out_type=jax.ShapeDtypeStruct((NC, N_PAD, D), jnp.float32),
    mesh=_mesh,
    compiler_params=_sc_params_flat,
    scratch_types=[
        pltpu.VMEM((G_CHUNKS, GK), jnp.int32),
        pltpu.VMEM((G_CHUNKS, GK), jnp.int32),
        pltpu.VMEM((3, GK, D), jnp.float32),
        pltpu.VMEM_SHARED((N_PAD, D), jnp.float32),
        pltpu.SemaphoreType.DMA,
        (pltpu.SemaphoreType.DMA, pltpu.SemaphoreType.DMA,
         pltpu.SemaphoreType.DMA),
        (pltpu.SemaphoreType.DMA, pltpu.SemaphoreType.DMA,
         pltpu.SemaphoreType.DMA),
    ],
)
def _aggregate(x_hbm, src_hbm, dst_hbm, out_hbm, sidx, didx, rows, acc_sh,
               isem, gsems, ssems):
    c = lax.axis_index("c")
    s = lax.axis_index("s")
    w = s * NC + c
    zerov = jnp.zeros((16,), jnp.float32)

    icp_s = pltpu.async_copy(src_hbm.at[w], sidx, isem)
    icp_d = pltpu.async_copy(dst_hbm.at[w], didx, isem)

    def zrow(i, _):
        for j in range(D // 16):
            rows[0, i, pl.ds(j * 16, 16)] = zerov
        return 0

    lax.fori_loop(0, GK, zrow, 0)
    base_r = s * ROWS_PER_TILE
    for t in range(ROWS_PER_TILE // GK):
        pltpu.sync_copy(rows.at[0], acc_sh.at[pl.ds(base_r + t * GK, GK)])
    icp_s.wait()
    icp_d.wait()
    plsc.subcore_barrier()

    def gather(k, b):
        pltpu.async_copy(x_hbm.at[sidx.at[k]], rows.at[b], gsems[b])

    def wait_gather(b):
        pltpu.make_async_copy(x_hbm.at[sidx.at[0]], rows.at[b], gsems[b]).wait()

    def scatter(k, b):
        pltpu.async_copy(rows.at[b], acc_sh.at[didx.at[k]], ssems[b], add=True)

    def wait_scatter(b):
        pltpu.make_async_copy(rows.at[b], acc_sh.at[didx.at[0]],
                              ssems[b]).wait()

    gather(0, 0)
    gather(1, 1)

    def tri(k3, _):
        for j in range(3):
            kk = 3 * k3 + j
            wait_gather(j)
            scatter(kk, j)
            nb = (j + 2) % 3

            @pl.when(kk >= 1)
            def _():
                wait_scatter(nb)

            gather(kk + 2, nb)
        return 0

    lax.fori_loop(0, (G_CHUNKS - 2) // 3, tri, 0)
    for kk in range(G_CHUNKS - 2, G_CHUNKS):
        b = kk % 3
        wait_gather(b)
        scatter(kk, b)
    for b in range(3):
        wait_scatter(b)
    plsc.subcore_barrier()
    pltpu.sync_copy(acc_sh.at[pl.ds(base_r, ROWS_PER_TILE)],
                    out_hbm.at[c, pl.ds(base_r, ROWS_PER_TILE)])


@functools.partial(
    pl.kernel,
    out_type=jax.ShapeDtypeStruct((OUT_DIM, EC_PAD), jnp.float32),
    mesh=_mesh,
    compiler_params=_sc_params,
    scratch_types=[
        pltpu.VMEM((N_PAD,), jnp.float32),
        pltpu.VMEM((N_PAD,), jnp.float32),
        pltpu.VMEM((N_PAD,), jnp.float32),
        pltpu.VMEM((N_PAD,), jnp.float32),
        pltpu.VMEM((16,), jnp.float32),
        pltpu.VMEM((C_CHUNKS, K), jnp.int32),
        pltpu.VMEM((C_CHUNKS, K), jnp.int32),
        pltpu.VMEM((K,), jnp.float32),
        pltpu.VMEM((K,), jnp.float32),
        pltpu.SemaphoreType.DMA,
    ],
)
def _classify(pq_hbm, bc_hbm, s_hbm, d_hbm, out_hbm,
              p0v, p1v, q0v, q1v, bcv, sidx, didx, st0, st1, isem):
    c = lax.axis_index("c")
    s = lax.axis_index("s")
    w = s * NC + c
    icp_s = pltpu.async_copy(s_hbm.at[w], sidx, isem)
    icp_d = pltpu.async_copy(d_hbm.at[w], didx, isem)
    pltpu.sync_copy(pq_hbm.at[0], p0v)
    pltpu.sync_copy(pq_hbm.at[1], p1v)
    pltpu.sync_copy(pq_hbm.at[2], q0v)
    pltpu.sync_copy(pq_hbm.at[3], q1v)
    pltpu.sync_copy(bc_hbm, bcv)
    zid = jnp.zeros((16,), jnp.int32)
    b0 = plsc.load_gather(bcv, [zid])
    b1 = plsc.load_gather(bcv, [zid + 1])
    icp_s.wait()
    icp_d.wait()

    def step(k, _):
        base = w * EC_PER_TILE + k * K
        for j in range(K // 16):
            sv = sidx[k, pl.ds(j * 16, 16)]
            dv = didx[k, pl.ds(j * 16, 16)]
            l0 = plsc.load_gather(p0v, [sv]) + plsc.load_gather(q0v, [dv]) + b0
            l1 = plsc.load_gather(p1v, [sv]) + plsc.load_gather(q1v, [dv]) + b1
            st0[pl.ds(j * 16, 16)] = 1.0 / (1.0 + jnp.exp(-l0))
            st1[pl.ds(j * 16, 16)] = 1.0 / (1.0 + jnp.exp(-l1))
        pltpu.sync_copy(st0, out_hbm.at[0, pl.ds(base, K)])
        pltpu.sync_copy(st1, out_hbm.at[1, pl.ds(base, K)])
        return 0

    lax.fori_loop(0, C_CHUNKS, step, 0)


BLK = 1024
NB = N_PAD // BLK
HB = BLK * HW // 128


def _deg_col(blk):
    d = blk[0] + blk[1]
    d = d.reshape(HB, 8, 16)[:, :, 0]
    return d.reshape(BLK, 1)


def _prep_body(hist_ref, feat_ref, x1_ref):
    ns = lax.rsqrt(jnp.maximum(_deg_col(hist_ref[...]), 1.0))
    x1_ref[...] = feat_ref[...] * ns


_prep = pl.pallas_call(
    _prep_body,
    grid=(NB,),
    in_specs=[
        pl.BlockSpec((NC, HB, 128), lambda i: (0, i, 0)),
        pl.BlockSpec((BLK, D), lambda i: (i, 0)),
    ],
    out_specs=pl.BlockSpec((BLK, D), lambda i: (i, 0)),
    out_shape=jax.ShapeDtypeStruct((N_PAD, D), jnp.float32),
)


def _layer1_body(agg_ref, hd_ref, hs_ref, w_ref, b_ref, out_ref):
    a = agg_ref[0] + agg_ref[1]
    nd = lax.rsqrt(jnp.maximum(_deg_col(hd_ref[...]), 1.0))
    y = jnp.dot(a * nd, w_ref[...], preferred_element_type=jnp.float32)
    y = jnp.maximum(y + b_ref[...], 0.0)
    ns = lax.rsqrt(jnp.maximum(_deg_col(hs_ref[...]), 1.0))
    out_ref[...] = y * ns


_layer1 = pl.pallas_call(
    _layer1_body,
    grid=(NB,),
    in_specs=[
        pl.BlockSpec((NC, BLK, D), lambda i: (0, i, 0)),
        pl.BlockSpec((NC, HB, 128), lambda i: (0, NB + i, 0)),
        pl.BlockSpec((NC, HB, 128), lambda i: (0, i, 0)),
        pl.BlockSpec((D, D), lambda i: (0, 0)),
        pl.BlockSpec((1, D), lambda i: (0, 0)),
    ],
    out_specs=pl.BlockSpec((BLK, D), lambda i: (i, 0)),
    out_shape=jax.ShapeDtypeStruct((N_PAD, D), jnp.float32),
)


def _layer2_body(agg_ref, hd_ref, w_ref, b_ref, wc_ref, h_ref, pq_ref):
    a = agg_ref[0] + agg_ref[1]
    nd = lax.rsqrt(jnp.maximum(_deg_col(hd_ref[...]), 1.0))
    y = jnp.dot(a * nd, w_ref[...], preferred_element_type=jnp.float32)
    y = jnp.maximum(y + b_ref[...], 0.0)
    h_ref[...] = y
    for j in range(4):
        pj = jnp.dot(y, wc_ref[...][:, j], preferred_element_type=jnp.float32)
        pq_ref[j] = pj.reshape(BLK // 128, 128)


_layer2 = pl.pallas_call(
    _layer2_body,
    grid=(NB,),
    in_specs=[
        pl.BlockSpec((NC, BLK, D), lambda i: (0, i, 0)),
        pl.BlockSpec((NC, HB, 128), lambda i: (0, NB + i, 0)),
        pl.BlockSpec((D, D), lambda i: (0, 0)),
        pl.BlockSpec((1, D), lambda i: (0, 0)),
        pl.BlockSpec((D, 8), lambda i: (0, 0)),
    ],
    out_specs=[
        pl.BlockSpec((BLK, D), lambda i: (i, 0)),
        pl.BlockSpec((4, BLK // 128, 128), lambda i: (0, i, 0)),
    ],
    out_shape=[
        jax.ShapeDtypeStruct((N_NODES, D), jnp.float32),
        jax.ShapeDtypeStruct((4, N_PAD // 128, 128), jnp.float32),
    ],
)


def kernel(feat, graph_edge_index, edge_index, W1, b1, W2, b2, Wc, bc):
    f32 = jnp.float32
    feat_pad = jnp.pad(feat.astype(f32), ((0, N_PAD - N_NODES), (0, 0)))

    n_pad_e = E_PAD - N_EDGES
    pad_idx = (jnp.arange(n_pad_e, dtype=jnp.int32) % (N_PAD - N_NODES)) + N_NODES
    src_p = jnp.concatenate([graph_edge_index[0].astype(jnp.int32), pad_idx])
    dst_p = jnp.concatenate([graph_edge_index[1].astype(jnp.int32), pad_idx])
    src3 = src_p.reshape(NW, G_CHUNKS, GK)
    dst3 = dst_p.reshape(NW, G_CHUNKS, GK)

    n_pad_c = EC_PAD - N_CLS
    zpad = jnp.zeros((n_pad_c,), jnp.int32)
    cs3 = jnp.concatenate([edge_index[0].astype(jnp.int32), zpad]).reshape(
        NW, C_CHUNKS, K)
    cd3 = jnp.concatenate([edge_index[1].astype(jnp.int32), zpad]).reshape(
        NW, C_CHUNKS, K)

    b1r = b1.astype(f32).reshape(1, D)
    b2r = b2.astype(f32).reshape(1, D)
    wc8 = jnp.zeros((D, 8), f32)
    wc8 = wc8.at[:, 0:2].set(Wc[:D].astype(f32))
    wc8 = wc8.at[:, 2:4].set(Wc[D:].astype(f32))
    bc16 = jnp.zeros((16,), f32).at[0:2].set(bc.astype(f32))

    hist = _degrees(src3, dst3)
    hist2 = hist.reshape(NC, 2 * N_PAD * HW // 128, 128)
    x1 = _prep(hist2, feat_pad)
    agg1 = _aggregate(x1, src3, dst3)
    x2 = _layer1(agg1, hist2, hist2, W1.astype(f32), b1r)
    agg2 = _aggregate(x2, src3, dst3)
    h, pq = _layer2(agg2, hist2, W2.astype(f32), b2r, wc8)
    planes = _classify(pq.reshape(4, N_PAD), bc16, cs3, cd3)

    probs = jnp.stack([planes[0, :N_CLS], planes[1, :N_CLS]], axis=1)
    return h, probs
